# baseline (speedup 1.0000x reference)
import jax
import jax.numpy as jnp
from jax.experimental import pallas as pl

_BN = 8192


def _body(v_ref, lo_ref, sinv_ref, dcat_ref, o_ref):
    v = v_ref[0]
    lo = lo_ref[...]
    sinv = sinv_ref[...]
    u = jnp.clip((v - lo) * sinv, 0.0, 1.0)
    ubits = jax.lax.bitcast_convert_type(u, jnp.uint32)
    uhi = jax.lax.bitcast_convert_type(
        ubits & jnp.uint32(0xFFFF0000), jnp.float32)
    ulo = u - uhi
    u1 = uhi.astype(jnp.bfloat16)
    u2 = ulo.astype(jnp.bfloat16)
    ucat = jnp.concatenate([u1, u1, u2], axis=0)
    o_ref[...] = jax.lax.dot_general(
        ucat, dcat_ref[...],
        dimension_numbers=(((0,), (0,)), ((), ())),
        preferred_element_type=jnp.float32,
    )


def kernel(values, boundaries, table):
    n = values.shape[0]
    nb = table.shape[0]
    d = jnp.concatenate([table[:1], table[1:] - table[:-1]], axis=0)
    dhi = jax.lax.bitcast_convert_type(
        jax.lax.bitcast_convert_type(d, jnp.uint32) & jnp.uint32(0xFFFF0000),
        jnp.float32)
    d1 = dhi.astype(jnp.bfloat16)
    d2 = (d - dhi).astype(jnp.bfloat16)
    dcat = jnp.concatenate([d1, d2, d1], axis=0)
    lo = jnp.concatenate(
        [jnp.full((1,), -3e30, jnp.float32), boundaries])[:, None]
    seg = boundaries[1:] - boundaries[:-1]
    sinv = jnp.concatenate(
        [jnp.ones((1,), jnp.float32), 1.0 / seg,
         jnp.full((1,), 1e30, jnp.float32)])[:, None]

    g = n // _BN
    v2 = values.reshape(g, 1, _BN)
    return pl.pallas_call(
        _body,
        grid=(g,),
        in_specs=[
            pl.BlockSpec((1, 1, _BN), lambda i: (i, 0, 0)),
            pl.BlockSpec((nb, 1), lambda i: (0, 0)),
            pl.BlockSpec((nb, 1), lambda i: (0, 0)),
            pl.BlockSpec((3 * nb, 64), lambda i: (0, 0)),
        ],
        out_specs=pl.BlockSpec((_BN, 64), lambda i: (i, 0)),
        out_shape=jax.ShapeDtypeStruct((n, 64), jnp.float32),
    )(v2, lo, sinv, dcat)


# speedup vs baseline: 1.0399x; 1.0399x over previous
import jax
import jax.numpy as jnp
from jax.experimental import pallas as pl

_BN = 16384


def _body(v_ref, lo_ref, sinv_ref, dcat_ref, o_ref):
    v = v_ref[0]
    lo = lo_ref[...]
    sinv = sinv_ref[...]
    u = jnp.clip((v - lo) * sinv, 0.0, 1.0)
    ubits = jax.lax.bitcast_convert_type(u, jnp.uint32)
    uhi = jax.lax.bitcast_convert_type(
        ubits & jnp.uint32(0xFFFF0000), jnp.float32)
    ulo = u - uhi
    u1 = uhi.astype(jnp.bfloat16)
    u2 = ulo.astype(jnp.bfloat16)
    ucat = jnp.concatenate([u1, u1, u2], axis=0)
    o_ref[...] = jax.lax.dot_general(
        ucat, dcat_ref[...],
        dimension_numbers=(((0,), (0,)), ((), ())),
        preferred_element_type=jnp.float32,
    )


def kernel(values, boundaries, table):
    n = values.shape[0]
    nb = table.shape[0]
    d = jnp.concatenate([table[:1], table[1:] - table[:-1]], axis=0)
    dhi = jax.lax.bitcast_convert_type(
        jax.lax.bitcast_convert_type(d, jnp.uint32) & jnp.uint32(0xFFFF0000),
        jnp.float32)
    d1 = dhi.astype(jnp.bfloat16)
    d2 = (d - dhi).astype(jnp.bfloat16)
    dcat = jnp.concatenate([d1, d2, d1], axis=0)
    lo = jnp.concatenate(
        [jnp.full((1,), -3e30, jnp.float32), boundaries])[:, None]
    seg = boundaries[1:] - boundaries[:-1]
    sinv = jnp.concatenate(
        [jnp.ones((1,), jnp.float32), 1.0 / seg,
         jnp.full((1,), 1e30, jnp.float32)])[:, None]

    g = n // _BN
    v2 = values.reshape(g, 1, _BN)
    return pl.pallas_call(
        _body,
        grid=(g,),
        in_specs=[
            pl.BlockSpec((1, 1, _BN), lambda i: (i, 0, 0)),
            pl.BlockSpec((nb, 1), lambda i: (0, 0)),
            pl.BlockSpec((nb, 1), lambda i: (0, 0)),
            pl.BlockSpec((3 * nb, 64), lambda i: (0, 0)),
        ],
        out_specs=pl.BlockSpec((_BN, 64), lambda i: (i, 0)),
        out_shape=jax.ShapeDtypeStruct((n, 64), jnp.float32),
    )(v2, lo, sinv, dcat)
